# baseline (device time: 12187 ns/iter reference)
import jax
import jax.numpy as jnp
from jax import lax
from jax.experimental import pallas as pl
from jax.experimental.pallas import tpu as pltpu

N_CHUNK = 4


def kernel(x):
    m, n = x.shape
    half = n // 2
    rpc = m // N_CHUNK

    def body(
        x_ref, out_ref,
        stage_f32, send_buf, local_f32, local_bf,
        load_sems, local_sem, out_sem, send_sems, recv_sems,
    ):
        my_x = lax.axis_index("x")
        my_y = lax.axis_index("y")
        my_z = lax.axis_index("z")
        peer_x = 1 - my_x
        peer = (peer_x, my_y, my_z)

        barrier_sem = pltpu.get_barrier_semaphore()
        pl.semaphore_signal(
            barrier_sem, inc=1,
            device_id=peer, device_id_type=pl.DeviceIdType.MESH,
        )

        loads = []
        for c in range(N_CHUNK):
            cp = pltpu.make_async_copy(
                x_ref.at[pl.ds(c * rpc, rpc), pl.ds(peer_x * half, half)],
                stage_f32.at[pl.ds(c * rpc, rpc), :],
                load_sems.at[c],
            )
            cp.start()
            loads.append(cp)
        local_cp = pltpu.make_async_copy(
            x_ref.at[:, pl.ds(my_x * half, half)], local_f32, local_sem
        )
        local_cp.start()

        pl.semaphore_wait(barrier_sem, 1)

        rdmas = []
        for c in range(N_CHUNK):
            loads[c].wait()
            send_buf[pl.ds(c * rpc, rpc), :] = stage_f32[
                pl.ds(c * rpc, rpc), :
            ].astype(jnp.bfloat16)
            rdma = pltpu.make_async_remote_copy(
                src_ref=send_buf.at[pl.ds(c * rpc, rpc), :],
                dst_ref=out_ref.at[pl.ds(my_x * m + c * rpc, rpc), :],
                send_sem=send_sems.at[c],
                recv_sem=recv_sems.at[c],
                device_id=peer,
                device_id_type=pl.DeviceIdType.MESH,
            )
            rdma.start()
            rdmas.append(rdma)

        local_cp.wait()
        local_bf[...] = local_f32[...].astype(jnp.bfloat16)
        out_cp = pltpu.make_async_copy(
            local_bf, out_ref.at[pl.ds(my_x * m, m), :], out_sem
        )
        out_cp.start()

        for rdma in rdmas:
            rdma.wait()
        out_cp.wait()

    return pl.pallas_call(
        body,
        out_shape=jax.ShapeDtypeStruct((2 * m, half), jnp.bfloat16),
        in_specs=[pl.BlockSpec(memory_space=pl.ANY)],
        out_specs=pl.BlockSpec(memory_space=pl.ANY),
        scratch_shapes=[
            pltpu.VMEM((m, half), jnp.float32),
            pltpu.VMEM((m, half), jnp.bfloat16),
            pltpu.VMEM((m, half), jnp.float32),
            pltpu.VMEM((m, half), jnp.bfloat16),
            pltpu.SemaphoreType.DMA((N_CHUNK,)),
            pltpu.SemaphoreType.DMA,
            pltpu.SemaphoreType.DMA,
            pltpu.SemaphoreType.DMA((N_CHUNK,)),
            pltpu.SemaphoreType.DMA((N_CHUNK,)),
        ],
        compiler_params=pltpu.CompilerParams(collective_id=0),
    )(x)
